# baseline (speedup 1.0000x reference)
import functools

import jax
import jax.numpy as jnp
from jax import lax
from jax.experimental import pallas as pl
from jax.experimental.pallas import tpu as pltpu
from jax.experimental.pallas import tpu_sc as plsc

NC = 2
NS = 16
NW = NC * NS
L = 16
CB = 1024


def _ceil_to(x, m):
    return (x + m - 1) // m * m


@functools.lru_cache(maxsize=None)
def _make_sc_call(T, N, E):
    assert T == L, "kernel assumes one timestep per vector lane"
    EW = _ceil_to(E, NW * CB) // NW
    NCHUNK = EW // CB
    EP = EW * NW
    NP = _ceil_to(N, NW * 8)
    RW = NP // NW

    mesh = plsc.VectorSubcoreMesh(core_axis_name="c", subcore_axis_name="s")

    def body(tabu, tabv, rows, cols, outpe, outke,
             idxr, idxc, gr, gc, vbuf, osc, sem1, sem2):
        wid = lax.axis_index("s") * NC + lax.axis_index("c")
        ebase = wid * EW
        zero = jnp.zeros((L,), jnp.float32)

        def chunk(i, acc):
            base = ebase + i * CB
            pltpu.sync_copy(rows.at[pl.ds(base, CB)], idxr)
            pltpu.sync_copy(cols.at[pl.ds(base, CB)], idxc)
            cp1 = pltpu.async_copy(tabu.at[idxr], gr, sem1)
            cp2 = pltpu.async_copy(tabu.at[idxc], gc, sem2)
            cp1.wait()
            cp2.wait()

            def edge8(j, accs):
                a0, a1 = accs
                e = j * 8
                d = gr[e] - gc[e]
                a0 = a0 + d * d
                d = gr[e + 1] - gc[e + 1]
                a1 = a1 + d * d
                d = gr[e + 2] - gc[e + 2]
                a0 = a0 + d * d
                d = gr[e + 3] - gc[e + 3]
                a1 = a1 + d * d
                d = gr[e + 4] - gc[e + 4]
                a0 = a0 + d * d
                d = gr[e + 5] - gc[e + 5]
                a1 = a1 + d * d
                d = gr[e + 6] - gc[e + 6]
                a0 = a0 + d * d
                d = gr[e + 7] - gc[e + 7]
                a1 = a1 + d * d
                return (a0, a1)

            accs = lax.fori_loop(0, CB // 8, edge8, (zero, zero))
            return acc + (accs[0] + accs[1])

        pe = lax.fori_loop(0, NCHUNK, chunk, zero)
        osc[...] = pe
        pltpu.sync_copy(osc, outpe.at[wid])

        pltpu.sync_copy(tabv.at[pl.ds(wid * RW, RW)], vbuf)

        def krow(r, a):
            vv = vbuf[r]
            return a + vv * vv

        ke = lax.fori_loop(0, RW, krow, zero, unroll=8)
        osc[...] = ke
        pltpu.sync_copy(osc, outke.at[wid])

    call = pl.kernel(
        body,
        out_type=(
            jax.ShapeDtypeStruct((NW, L), jnp.float32),
            jax.ShapeDtypeStruct((NW, L), jnp.float32),
        ),
        mesh=mesh,
        scratch_types=[
            pltpu.VMEM((CB,), jnp.int32),
            pltpu.VMEM((CB,), jnp.int32),
            pltpu.VMEM((CB, L), jnp.float32),
            pltpu.VMEM((CB, L), jnp.float32),
            pltpu.VMEM((RW, L), jnp.float32),
            pltpu.VMEM((L,), jnp.float32),
            pltpu.SemaphoreType.DMA,
            pltpu.SemaphoreType.DMA,
        ],
        compiler_params=pltpu.CompilerParams(use_tc_tiling_on_sc=False),
    )
    return call, EP, NP


def kernel(states, edge_index):
    T, N, _ = states.shape
    E = edge_index.shape[1]
    call, EP, NP = _make_sc_call(T, N, E)

    tabu = states[:, :, 0].T
    tabv = jnp.pad(states[:, :, 1].T, ((0, NP - N), (0, 0)))
    ei = edge_index.astype(jnp.int32)
    eip = jnp.pad(ei, ((0, 0), (0, EP - E)))
    outpe, outke = call(tabu, tabv, eip[0], eip[1])

    H = 0.5 * (jnp.sum(outpe, axis=0) + jnp.sum(outke, axis=0))
    dH = H[1:] - H[:-1]
    return jnp.sum(dH * dH) / (T - 1)


# speedup vs baseline: 1.0163x; 1.0163x over previous
import functools

import jax
import jax.numpy as jnp
from jax import lax
from jax.experimental import pallas as pl
from jax.experimental.pallas import tpu as pltpu
from jax.experimental.pallas import tpu_sc as plsc

NC = 2
NS = 16
NW = NC * NS
L = 16
CB = 1000


def _ceil_to(x, m):
    return (x + m - 1) // m * m


@functools.lru_cache(maxsize=None)
def _make_sc_call(T, N, E):
    assert T == L, "kernel assumes one timestep per vector lane"
    EW = _ceil_to(E, NW * CB) // NW
    NCHUNK = EW // CB
    EP = EW * NW
    NP = _ceil_to(N, NW * 8)
    RW = NP // NW

    mesh = plsc.VectorSubcoreMesh(core_axis_name="c", subcore_axis_name="s")

    def body(tabu, tabv, rows, cols, outpe, outke,
             idxr, idxc, gr, gc, vbuf, osc, sem1, sem2):
        wid = lax.axis_index("s") * NC + lax.axis_index("c")
        ebase = wid * EW
        zero = jnp.zeros((L,), jnp.float32)

        def chunk(i, acc):
            base = ebase + i * CB
            pltpu.sync_copy(rows.at[pl.ds(base, CB)], idxr)
            pltpu.sync_copy(cols.at[pl.ds(base, CB)], idxc)
            cp1 = pltpu.async_copy(tabu.at[idxr], gr, sem1)
            cp2 = pltpu.async_copy(tabu.at[idxc], gc, sem2)
            cp1.wait()
            cp2.wait()

            def edge8(j, accs):
                a0, a1 = accs
                e = j * 8
                d = gr[e] - gc[e]
                a0 = a0 + d * d
                d = gr[e + 1] - gc[e + 1]
                a1 = a1 + d * d
                d = gr[e + 2] - gc[e + 2]
                a0 = a0 + d * d
                d = gr[e + 3] - gc[e + 3]
                a1 = a1 + d * d
                d = gr[e + 4] - gc[e + 4]
                a0 = a0 + d * d
                d = gr[e + 5] - gc[e + 5]
                a1 = a1 + d * d
                d = gr[e + 6] - gc[e + 6]
                a0 = a0 + d * d
                d = gr[e + 7] - gc[e + 7]
                a1 = a1 + d * d
                return (a0, a1)

            accs = lax.fori_loop(0, CB // 8, edge8, (zero, zero))
            return acc + (accs[0] + accs[1])

        assert CB % 8 == 0 and EW % CB == 0 and EW % 8 == 0

        pe = lax.fori_loop(0, NCHUNK, chunk, zero)
        osc[...] = pe
        pltpu.sync_copy(osc, outpe.at[wid])

        pltpu.sync_copy(tabv.at[pl.ds(wid * RW, RW)], vbuf)

        def krow(r, a):
            vv = vbuf[r]
            return a + vv * vv

        ke = lax.fori_loop(0, RW, krow, zero, unroll=8)
        osc[...] = ke
        pltpu.sync_copy(osc, outke.at[wid])

    call = pl.kernel(
        body,
        out_type=(
            jax.ShapeDtypeStruct((NW, L), jnp.float32),
            jax.ShapeDtypeStruct((NW, L), jnp.float32),
        ),
        mesh=mesh,
        scratch_types=[
            pltpu.VMEM((CB,), jnp.int32),
            pltpu.VMEM((CB,), jnp.int32),
            pltpu.VMEM((CB, L), jnp.float32),
            pltpu.VMEM((CB, L), jnp.float32),
            pltpu.VMEM((RW, L), jnp.float32),
            pltpu.VMEM((L,), jnp.float32),
            pltpu.SemaphoreType.DMA,
            pltpu.SemaphoreType.DMA,
        ],
        compiler_params=pltpu.CompilerParams(use_tc_tiling_on_sc=False),
    )
    return call, EP, NP


def kernel(states, edge_index):
    T, N, _ = states.shape
    E = edge_index.shape[1]
    call, EP, NP = _make_sc_call(T, N, E)

    tabu = states[:, :, 0].T
    tabv = jnp.pad(states[:, :, 1].T, ((0, NP - N), (0, 0)))
    ei = edge_index.astype(jnp.int32)
    assert EP == E, "CB must divide E/NW so no edge padding is needed"
    outpe, outke = call(tabu, tabv, ei[0], ei[1])

    H = 0.5 * (jnp.sum(outpe, axis=0) + jnp.sum(outke, axis=0))
    dH = H[1:] - H[:-1]
    return jnp.sum(dH * dH) / (T - 1)


# speedup vs baseline: 1.1786x; 1.1597x over previous
import functools

import jax
import jax.numpy as jnp
from jax import lax
from jax.experimental import pallas as pl
from jax.experimental.pallas import tpu as pltpu
from jax.experimental.pallas import tpu_sc as plsc

NC = 2
NS = 16
NW = NC * NS
L = 16
CB = 1000


def _ceil_to(x, m):
    return (x + m - 1) // m * m


@functools.lru_cache(maxsize=None)
def _make_sc_call(T, N, E):
    assert T == L, "kernel assumes one timestep per vector lane"
    EW = E // NW
    NCHUNK = EW // CB
    assert EW % CB == 0 and CB % 8 == 0 and EW % 8 == 0
    RW = _ceil_to(N, NW * 8) // NW
    assert RW % 8 == 0 and (NW * RW - N) % 8 == 0 and N >= RW

    mesh = plsc.VectorSubcoreMesh(core_axis_name="c", subcore_axis_name="s")

    def body(tabu, tabv, ei, outh, idxr, idxc, gr, gc, vbuf, osc,
             sem1, sem2, semv):
        wid = lax.axis_index("s") * NC + lax.axis_index("c")
        ebase = wid * EW
        zero = jnp.zeros((L,), jnp.float32)

        vbase = jnp.minimum(wid * RW, N - RW)
        vskip = wid * RW - vbase
        pltpu.async_copy(tabv.at[pl.ds(vbase, RW)], vbuf, semv)

        def chunk(i, acc):
            base = ebase + i * CB
            pltpu.sync_copy(ei.at[0, pl.ds(base, CB)], idxr)
            pltpu.sync_copy(ei.at[1, pl.ds(base, CB)], idxc)
            cp1 = pltpu.async_copy(tabu.at[idxr], gr, sem1)
            cp2 = pltpu.async_copy(tabu.at[idxc], gc, sem2)
            cp1.wait()
            cp2.wait()

            def edge8(j, accs):
                a0, a1 = accs
                e = j * 8
                d = gr[e] - gc[e]
                a0 = a0 + d * d
                d = gr[e + 1] - gc[e + 1]
                a1 = a1 + d * d
                d = gr[e + 2] - gc[e + 2]
                a0 = a0 + d * d
                d = gr[e + 3] - gc[e + 3]
                a1 = a1 + d * d
                d = gr[e + 4] - gc[e + 4]
                a0 = a0 + d * d
                d = gr[e + 5] - gc[e + 5]
                a1 = a1 + d * d
                d = gr[e + 6] - gc[e + 6]
                a0 = a0 + d * d
                d = gr[e + 7] - gc[e + 7]
                a1 = a1 + d * d
                return (a0, a1)

            accs = lax.fori_loop(0, CB // 8, edge8, (zero, zero))
            return acc + (accs[0] + accs[1])

        pe = lax.fori_loop(0, NCHUNK, chunk, zero)

        pltpu.make_async_copy(tabv.at[pl.ds(vbase, RW)], vbuf, semv).wait()

        def krow8(j, a):
            r = j * 8
            x = vbuf[r]
            a = a + x * x
            x = vbuf[r + 1]
            a = a + x * x
            x = vbuf[r + 2]
            a = a + x * x
            x = vbuf[r + 3]
            a = a + x * x
            x = vbuf[r + 4]
            a = a + x * x
            x = vbuf[r + 5]
            a = a + x * x
            x = vbuf[r + 6]
            a = a + x * x
            x = vbuf[r + 7]
            a = a + x * x
            return a

        ke = lax.fori_loop(vskip // 8, RW // 8, krow8, zero)
        osc[...] = pe + ke
        pltpu.sync_copy(osc, outh.at[wid])

    call = pl.kernel(
        body,
        out_type=jax.ShapeDtypeStruct((NW, L), jnp.float32),
        mesh=mesh,
        scratch_types=[
            pltpu.VMEM((CB,), jnp.int32),
            pltpu.VMEM((CB,), jnp.int32),
            pltpu.VMEM((CB, L), jnp.float32),
            pltpu.VMEM((CB, L), jnp.float32),
            pltpu.VMEM((RW, L), jnp.float32),
            pltpu.VMEM((L,), jnp.float32),
            pltpu.SemaphoreType.DMA,
            pltpu.SemaphoreType.DMA,
            pltpu.SemaphoreType.DMA,
        ],
        compiler_params=pltpu.CompilerParams(use_tc_tiling_on_sc=False),
    )
    return call


def kernel(states, edge_index):
    T, N, _ = states.shape
    E = edge_index.shape[1]
    call = _make_sc_call(T, N, E)

    tabu = states[:, :, 0].T
    tabv = states[:, :, 1].T
    ei = edge_index.astype(jnp.int32)
    outh = call(tabu, tabv, ei)

    H = 0.5 * jnp.sum(outh, axis=0)
    dH = H[1:] - H[:-1]
    return jnp.sum(dH * dH) / (T - 1)


# speedup vs baseline: 1.3086x; 1.1103x over previous
import functools

import jax
import jax.numpy as jnp
from jax import lax
from jax.experimental import pallas as pl
from jax.experimental.pallas import tpu as pltpu
from jax.experimental.pallas import tpu_sc as plsc

NC = 2
NS = 16
NW = NC * NS
L = 16
CB = 2000


def _ceil_to(x, m):
    return (x + m - 1) // m * m


@functools.lru_cache(maxsize=None)
def _make_sc_call(T, N, E):
    assert T == L, "kernel assumes one timestep per vector lane"
    EW = E // NW
    NCHUNK = EW // CB
    assert EW % CB == 0 and CB % 8 == 0 and EW % 8 == 0
    RW = _ceil_to(N, NW * 8) // NW
    assert RW % 8 == 0 and (NW * RW - N) % 8 == 0 and N >= RW

    mesh = plsc.VectorSubcoreMesh(core_axis_name="c", subcore_axis_name="s")

    def body(tabu, tabv, ei, outh, idxr, idxc, gr, gc, vbuf, osc,
             sem1, sem2, semv):
        wid = lax.axis_index("s") * NC + lax.axis_index("c")
        ebase = wid * EW
        zero = jnp.zeros((L,), jnp.float32)

        vbase = jnp.minimum(wid * RW, N - RW)
        vskip = wid * RW - vbase
        pltpu.async_copy(tabv.at[pl.ds(vbase, RW)], vbuf, semv)

        def chunk(i, acc):
            base = ebase + i * CB
            pltpu.sync_copy(ei.at[0, pl.ds(base, CB)], idxr)
            pltpu.sync_copy(ei.at[1, pl.ds(base, CB)], idxc)
            cp1 = pltpu.async_copy(tabu.at[idxr], gr, sem1)
            cp2 = pltpu.async_copy(tabu.at[idxc], gc, sem2)
            cp1.wait()
            cp2.wait()

            def edge8(j, accs):
                a0, a1 = accs
                e = j * 8
                d = gr[e] - gc[e]
                a0 = a0 + d * d
                d = gr[e + 1] - gc[e + 1]
                a1 = a1 + d * d
                d = gr[e + 2] - gc[e + 2]
                a0 = a0 + d * d
                d = gr[e + 3] - gc[e + 3]
                a1 = a1 + d * d
                d = gr[e + 4] - gc[e + 4]
                a0 = a0 + d * d
                d = gr[e + 5] - gc[e + 5]
                a1 = a1 + d * d
                d = gr[e + 6] - gc[e + 6]
                a0 = a0 + d * d
                d = gr[e + 7] - gc[e + 7]
                a1 = a1 + d * d
                return (a0, a1)

            accs = lax.fori_loop(0, CB // 8, edge8, (zero, zero))
            return acc + (accs[0] + accs[1])

        pe = lax.fori_loop(0, NCHUNK, chunk, zero)

        pltpu.make_async_copy(tabv.at[pl.ds(vbase, RW)], vbuf, semv).wait()

        def krow8(j, a):
            r = j * 8
            x = vbuf[r]
            a = a + x * x
            x = vbuf[r + 1]
            a = a + x * x
            x = vbuf[r + 2]
            a = a + x * x
            x = vbuf[r + 3]
            a = a + x * x
            x = vbuf[r + 4]
            a = a + x * x
            x = vbuf[r + 5]
            a = a + x * x
            x = vbuf[r + 6]
            a = a + x * x
            x = vbuf[r + 7]
            a = a + x * x
            return a

        ke = lax.fori_loop(vskip // 8, RW // 8, krow8, zero)
        osc[...] = pe + ke
        pltpu.sync_copy(osc, outh.at[wid])

    call = pl.kernel(
        body,
        out_type=jax.ShapeDtypeStruct((NW, L), jnp.float32),
        mesh=mesh,
        scratch_types=[
            pltpu.VMEM((CB,), jnp.int32),
            pltpu.VMEM((CB,), jnp.int32),
            pltpu.VMEM((CB, L), jnp.float32),
            pltpu.VMEM((CB, L), jnp.float32),
            pltpu.VMEM((RW, L), jnp.float32),
            pltpu.VMEM((L,), jnp.float32),
            pltpu.SemaphoreType.DMA,
            pltpu.SemaphoreType.DMA,
            pltpu.SemaphoreType.DMA,
        ],
        compiler_params=pltpu.CompilerParams(use_tc_tiling_on_sc=False),
    )
    return call


def kernel(states, edge_index):
    T, N, _ = states.shape
    E = edge_index.shape[1]
    call = _make_sc_call(T, N, E)

    tabu = states[:, :, 0].T
    tabv = states[:, :, 1].T
    ei = edge_index.astype(jnp.int32)
    outh = call(tabu, tabv, ei)

    H = 0.5 * jnp.sum(outh, axis=0)
    dH = H[1:] - H[:-1]
    return jnp.sum(dH * dH) / (T - 1)


# speedup vs baseline: 1.5619x; 1.1935x over previous
import functools

import jax
import jax.numpy as jnp
from jax import lax
from jax.experimental import pallas as pl
from jax.experimental.pallas import tpu as pltpu
from jax.experimental.pallas import tpu_sc as plsc

NC = 2
NS = 16
NW = NC * NS
L = 16
CB = 1000


def _ceil_to(x, m):
    return (x + m - 1) // m * m


@functools.lru_cache(maxsize=None)
def _make_sc_call(T, N, E):
    assert T == L, "kernel assumes one timestep per vector lane"
    EW = E // NW
    NCHUNK = EW // CB
    assert EW % CB == 0 and CB % 8 == 0 and EW % 8 == 0
    RW = _ceil_to(N, NW * 8) // NW
    assert RW % 8 == 0 and (NW * RW - N) % 8 == 0 and N >= RW

    mesh = plsc.VectorSubcoreMesh(core_axis_name="c", subcore_axis_name="s")

    def body(tabu, tabv, ei, outh, idx, gbuf, vbuf, osc,
             semg, semv):
        wid = lax.axis_index("s") * NC + lax.axis_index("c")
        ebase = wid * EW
        zero = jnp.zeros((L,), jnp.float32)

        vbase = jnp.minimum(wid * RW, N - RW)
        vskip = wid * RW - vbase
        pltpu.async_copy(tabv.at[pl.ds(vbase, RW)], vbuf, semv)

        def load_fire(c, s):
            base = ebase + c * CB
            pltpu.sync_copy(ei.at[0, pl.ds(base, CB)], idx.at[s, 0])
            pltpu.sync_copy(ei.at[1, pl.ds(base, CB)], idx.at[s, 1])
            pltpu.async_copy(tabu.at[idx.at[s, 0]], gbuf.at[s, 0], semg.at[s])
            pltpu.async_copy(tabu.at[idx.at[s, 1]], gbuf.at[s, 1], semg.at[s])

        load_fire(0, 0)

        def chunk(i, acc):
            s = lax.rem(i, 2)

            @pl.when(i + 1 < NCHUNK)
            def _():
                load_fire(i + 1, 1 - s)

            pltpu.make_async_copy(
                tabu.at[idx.at[s, 0]], gbuf.at[s, 0], semg.at[s]).wait()
            pltpu.make_async_copy(
                tabu.at[idx.at[s, 1]], gbuf.at[s, 1], semg.at[s]).wait()

            def edge8(j, accs):
                a0, a1 = accs
                e = j * 8
                d = gbuf[s, 0, e] - gbuf[s, 1, e]
                a0 = a0 + d * d
                d = gbuf[s, 0, e + 1] - gbuf[s, 1, e + 1]
                a1 = a1 + d * d
                d = gbuf[s, 0, e + 2] - gbuf[s, 1, e + 2]
                a0 = a0 + d * d
                d = gbuf[s, 0, e + 3] - gbuf[s, 1, e + 3]
                a1 = a1 + d * d
                d = gbuf[s, 0, e + 4] - gbuf[s, 1, e + 4]
                a0 = a0 + d * d
                d = gbuf[s, 0, e + 5] - gbuf[s, 1, e + 5]
                a1 = a1 + d * d
                d = gbuf[s, 0, e + 6] - gbuf[s, 1, e + 6]
                a0 = a0 + d * d
                d = gbuf[s, 0, e + 7] - gbuf[s, 1, e + 7]
                a1 = a1 + d * d
                return (a0, a1)

            accs = lax.fori_loop(0, CB // 8, edge8, (zero, zero))
            return acc + (accs[0] + accs[1])

        pe = lax.fori_loop(0, NCHUNK, chunk, zero)

        pltpu.make_async_copy(tabv.at[pl.ds(vbase, RW)], vbuf, semv).wait()

        def krow8(j, a):
            r = j * 8
            x = vbuf[r]
            a = a + x * x
            x = vbuf[r + 1]
            a = a + x * x
            x = vbuf[r + 2]
            a = a + x * x
            x = vbuf[r + 3]
            a = a + x * x
            x = vbuf[r + 4]
            a = a + x * x
            x = vbuf[r + 5]
            a = a + x * x
            x = vbuf[r + 6]
            a = a + x * x
            x = vbuf[r + 7]
            a = a + x * x
            return a

        ke = lax.fori_loop(vskip // 8, RW // 8, krow8, zero)
        osc[...] = pe + ke
        pltpu.sync_copy(osc, outh.at[wid])

    call = pl.kernel(
        body,
        out_type=jax.ShapeDtypeStruct((NW, L), jnp.float32),
        mesh=mesh,
        scratch_types=[
            pltpu.VMEM((2, 2, CB), jnp.int32),
            pltpu.VMEM((2, 2, CB, L), jnp.float32),
            pltpu.VMEM((RW, L), jnp.float32),
            pltpu.VMEM((L,), jnp.float32),
            pltpu.SemaphoreType.DMA((2,)),
            pltpu.SemaphoreType.DMA,
        ],
        compiler_params=pltpu.CompilerParams(use_tc_tiling_on_sc=False),
    )
    return call


def kernel(states, edge_index):
    T, N, _ = states.shape
    E = edge_index.shape[1]
    call = _make_sc_call(T, N, E)

    tabu = states[:, :, 0].T
    tabv = states[:, :, 1].T
    ei = edge_index.astype(jnp.int32)
    outh = call(tabu, tabv, ei)

    H = 0.5 * jnp.sum(outh, axis=0)
    dH = H[1:] - H[:-1]
    return jnp.sum(dH * dH) / (T - 1)


# speedup vs baseline: 1.6330x; 1.0456x over previous
import functools

import jax
import jax.numpy as jnp
from jax import lax
from jax.experimental import pallas as pl
from jax.experimental.pallas import tpu as pltpu
from jax.experimental.pallas import tpu_sc as plsc

NC = 2
NS = 16
NW = NC * NS
L = 16
CB = 1000


def _ceil_to(x, m):
    return (x + m - 1) // m * m


@functools.lru_cache(maxsize=None)
def _make_sc_call(T, N, E):
    assert T == L, "kernel assumes one timestep per vector lane"
    EW = E // NW
    NCHUNK = EW // CB
    assert EW % CB == 0 and CB % 8 == 0 and EW % 8 == 0
    RW = _ceil_to(N, NW * 8) // NW
    assert RW % 8 == 0 and (NW * RW - N) % 8 == 0 and N >= RW

    mesh = plsc.VectorSubcoreMesh(core_axis_name="c", subcore_axis_name="s")

    def body(tabu, tabv, ei, outh, idx, gbuf, vbuf, osc,
             semg, semv):
        wid = lax.axis_index("s") * NC + lax.axis_index("c")
        ebase = wid * EW
        zero = jnp.zeros((L,), jnp.float32)

        vbase = jnp.minimum(wid * RW, N - RW)
        vskip = wid * RW - vbase
        pltpu.async_copy(tabv.at[pl.ds(vbase, RW)], vbuf, semv)

        def load_fire(c, s):
            base = ebase + c * CB
            pltpu.sync_copy(ei.at[0, pl.ds(base, CB)], idx.at[s, 0])
            pltpu.sync_copy(ei.at[1, pl.ds(base, CB)], idx.at[s, 1])
            pltpu.async_copy(tabu.at[idx.at[s, 0]], gbuf.at[s, 0], semg.at[s])
            pltpu.async_copy(tabu.at[idx.at[s, 1]], gbuf.at[s, 1], semg.at[s])

        load_fire(0, 0)
        load_fire(1, 1)

        def chunk(i, acc):
            s = lax.rem(i, 3)

            @pl.when(i + 2 < NCHUNK)
            def _():
                load_fire(i + 2, lax.rem(i + 2, 3))

            pltpu.make_async_copy(
                tabu.at[idx.at[s, 0]], gbuf.at[s, 0], semg.at[s]).wait()
            pltpu.make_async_copy(
                tabu.at[idx.at[s, 1]], gbuf.at[s, 1], semg.at[s]).wait()

            def edge8(j, accs):
                a0, a1 = accs
                e = j * 8
                d = gbuf[s, 0, e] - gbuf[s, 1, e]
                a0 = a0 + d * d
                d = gbuf[s, 0, e + 1] - gbuf[s, 1, e + 1]
                a1 = a1 + d * d
                d = gbuf[s, 0, e + 2] - gbuf[s, 1, e + 2]
                a0 = a0 + d * d
                d = gbuf[s, 0, e + 3] - gbuf[s, 1, e + 3]
                a1 = a1 + d * d
                d = gbuf[s, 0, e + 4] - gbuf[s, 1, e + 4]
                a0 = a0 + d * d
                d = gbuf[s, 0, e + 5] - gbuf[s, 1, e + 5]
                a1 = a1 + d * d
                d = gbuf[s, 0, e + 6] - gbuf[s, 1, e + 6]
                a0 = a0 + d * d
                d = gbuf[s, 0, e + 7] - gbuf[s, 1, e + 7]
                a1 = a1 + d * d
                return (a0, a1)

            accs = lax.fori_loop(0, CB // 8, edge8, (zero, zero))
            return acc + (accs[0] + accs[1])

        pe = lax.fori_loop(0, NCHUNK, chunk, zero)

        pltpu.make_async_copy(tabv.at[pl.ds(vbase, RW)], vbuf, semv).wait()

        def krow8(j, a):
            r = j * 8
            x = vbuf[r]
            a = a + x * x
            x = vbuf[r + 1]
            a = a + x * x
            x = vbuf[r + 2]
            a = a + x * x
            x = vbuf[r + 3]
            a = a + x * x
            x = vbuf[r + 4]
            a = a + x * x
            x = vbuf[r + 5]
            a = a + x * x
            x = vbuf[r + 6]
            a = a + x * x
            x = vbuf[r + 7]
            a = a + x * x
            return a

        ke = lax.fori_loop(vskip // 8, RW // 8, krow8, zero)
        osc[...] = pe + ke
        pltpu.sync_copy(osc, outh.at[wid])

    call = pl.kernel(
        body,
        out_type=jax.ShapeDtypeStruct((NW, L), jnp.float32),
        mesh=mesh,
        scratch_types=[
            pltpu.VMEM((3, 2, CB), jnp.int32),
            pltpu.VMEM((3, 2, CB, L), jnp.float32),
            pltpu.VMEM((RW, L), jnp.float32),
            pltpu.VMEM((L,), jnp.float32),
            pltpu.SemaphoreType.DMA((3,)),
            pltpu.SemaphoreType.DMA,
        ],
        compiler_params=pltpu.CompilerParams(use_tc_tiling_on_sc=False),
    )
    return call


def kernel(states, edge_index):
    T, N, _ = states.shape
    E = edge_index.shape[1]
    call = _make_sc_call(T, N, E)

    tabu = states[:, :, 0].T
    tabv = states[:, :, 1].T
    ei = edge_index.astype(jnp.int32)
    outh = call(tabu, tabv, ei)

    H = 0.5 * jnp.sum(outh, axis=0)
    dH = H[1:] - H[:-1]
    return jnp.sum(dH * dH) / (T - 1)


# speedup vs baseline: 1.9537x; 1.1963x over previous
import functools

import jax
import jax.numpy as jnp
from jax import lax
from jax.experimental import pallas as pl
from jax.experimental.pallas import tpu as pltpu
from jax.experimental.pallas import tpu_sc as plsc

NC = 2
NS = 16
NW = NC * NS
L = 16
CB = 1000


def _ceil_to(x, m):
    return (x + m - 1) // m * m


@functools.lru_cache(maxsize=None)
def _make_sc_call(T, N, E):
    assert T == L, "kernel assumes one timestep per vector lane"
    EW = E // NW
    NCHUNK = EW // CB
    assert EW % CB == 0 and CB % 8 == 0 and EW % 8 == 0
    RW = _ceil_to(N, NW * 8) // NW
    assert RW % 8 == 0 and (NW * RW - N) % 8 == 0 and N >= RW

    mesh = plsc.VectorSubcoreMesh(core_axis_name="c", subcore_axis_name="s")

    def body(tabuv, ei, outh, idx, gbuf, vbuf, osc,
             semg, semv):
        tabu = tabuv.at[0]
        tabv = tabuv.at[1]
        wid = lax.axis_index("s") * NC + lax.axis_index("c")
        ebase = wid * EW
        zero = jnp.zeros((L,), jnp.float32)

        vbase = jnp.minimum(wid * RW, N - RW)
        vskip = wid * RW - vbase
        pltpu.async_copy(tabv.at[pl.ds(vbase, RW)], vbuf, semv)

        def load_fire(c, s):
            base = ebase + c * CB
            pltpu.sync_copy(ei.at[0, pl.ds(base, CB)], idx.at[s, 0])
            pltpu.sync_copy(ei.at[1, pl.ds(base, CB)], idx.at[s, 1])
            pltpu.async_copy(tabu.at[idx.at[s, 0]], gbuf.at[s, 0], semg.at[s])
            pltpu.async_copy(tabu.at[idx.at[s, 1]], gbuf.at[s, 1], semg.at[s])

        load_fire(0, 0)
        load_fire(1, 1)

        def chunk(i, acc):
            s = lax.rem(i, 3)

            @pl.when(i + 2 < NCHUNK)
            def _():
                load_fire(i + 2, lax.rem(i + 2, 3))

            pltpu.make_async_copy(
                tabu.at[idx.at[s, 0]], gbuf.at[s, 0], semg.at[s]).wait()
            pltpu.make_async_copy(
                tabu.at[idx.at[s, 1]], gbuf.at[s, 1], semg.at[s]).wait()

            def edge8(j, accs):
                a0, a1 = accs
                e = j * 8
                d = gbuf[s, 0, e] - gbuf[s, 1, e]
                a0 = a0 + d * d
                d = gbuf[s, 0, e + 1] - gbuf[s, 1, e + 1]
                a1 = a1 + d * d
                d = gbuf[s, 0, e + 2] - gbuf[s, 1, e + 2]
                a0 = a0 + d * d
                d = gbuf[s, 0, e + 3] - gbuf[s, 1, e + 3]
                a1 = a1 + d * d
                d = gbuf[s, 0, e + 4] - gbuf[s, 1, e + 4]
                a0 = a0 + d * d
                d = gbuf[s, 0, e + 5] - gbuf[s, 1, e + 5]
                a1 = a1 + d * d
                d = gbuf[s, 0, e + 6] - gbuf[s, 1, e + 6]
                a0 = a0 + d * d
                d = gbuf[s, 0, e + 7] - gbuf[s, 1, e + 7]
                a1 = a1 + d * d
                return (a0, a1)

            accs = lax.fori_loop(0, CB // 8, edge8, (zero, zero))
            return acc + (accs[0] + accs[1])

        pe = lax.fori_loop(0, NCHUNK, chunk, zero)

        pltpu.make_async_copy(tabv.at[pl.ds(vbase, RW)], vbuf, semv).wait()

        def krow8(j, a):
            r = j * 8
            x = vbuf[r]
            a = a + x * x
            x = vbuf[r + 1]
            a = a + x * x
            x = vbuf[r + 2]
            a = a + x * x
            x = vbuf[r + 3]
            a = a + x * x
            x = vbuf[r + 4]
            a = a + x * x
            x = vbuf[r + 5]
            a = a + x * x
            x = vbuf[r + 6]
            a = a + x * x
            x = vbuf[r + 7]
            a = a + x * x
            return a

        ke = lax.fori_loop(vskip // 8, RW // 8, krow8, zero)
        osc[...] = pe + ke
        pltpu.sync_copy(osc, outh.at[wid])

    call = pl.kernel(
        body,
        out_type=jax.ShapeDtypeStruct((NW, L), jnp.float32),
        mesh=mesh,
        scratch_types=[
            pltpu.VMEM((3, 2, CB), jnp.int32),
            pltpu.VMEM((3, 2, CB, L), jnp.float32),
            pltpu.VMEM((RW, L), jnp.float32),
            pltpu.VMEM((L,), jnp.float32),
            pltpu.SemaphoreType.DMA((3,)),
            pltpu.SemaphoreType.DMA,
        ],
        compiler_params=pltpu.CompilerParams(use_tc_tiling_on_sc=False),
    )
    return call


def kernel(states, edge_index):
    T, N, _ = states.shape
    E = edge_index.shape[1]
    call = _make_sc_call(T, N, E)

    tabuv = jnp.transpose(states, (2, 1, 0))
    ei = edge_index.astype(jnp.int32)
    outh = call(tabuv, ei)

    H = 0.5 * jnp.sum(outh, axis=0)
    dH = H[1:] - H[:-1]
    return jnp.sum(dH * dH) / (T - 1)
